# baseline (speedup 1.0000x reference)
import functools

import jax
import jax.numpy as jnp
from jax import lax
from jax.experimental import pallas as pl
from jax.experimental.pallas import tpu as pltpu
from jax.experimental.pallas import tpu_sc as plsc

N = 10000
D = 128
E = 320000

NC = 2
NS = 16
NW = NC * NS

NP = 10240
NPR = NP // 128
CK = 80
CH = 128
EPW = CH * CK
EP = NW * EPW
RPT = NP // NS
BLK = 2048
GRID = NP // BLK


def _zero_rows(ref, nrows):
    z = jnp.zeros((16,), jnp.float32)

    def body(r, _):
        for k in range(8):
            ref[r, pl.ds(k * 16, 16)] = z
        return 0

    lax.fori_loop(0, nrows, body, 0)


def _hist_body(col_hbm, dinv_hbm, invdeg_hbm, colv, lh, iotav, dbuf, obuf, sh):
    c = lax.axis_index("c")
    s = lax.axis_index("s")

    _zero_rows(lh, NPR)
    for k in range(NPR // 16):
        iotav[0, pl.ds(k * 16, 16)] = lax.iota(jnp.int32, 16) + 16 * k

    @pl.when(s < NPR // 8)
    def _():
        pltpu.sync_copy(lh.at[pl.ds(s * 8, 8)], sh.at[pl.ds(s * 8, 8)])

    plsc.subcore_barrier()

    ones = jnp.ones((16,), jnp.float32)

    for t in range(NW // NS):
        pltpu.sync_copy(col_hbm.at[s + NS * t], colv)

        def body(j, _):
            for i in range(CK // 16):
                idx = colv[j, pl.ds(i * 16, 16)]
                plsc.addupdate_scatter(
                    lh,
                    [lax.shift_right_logical(idx, 7),
                     lax.bitwise_and(idx, 127)],
                    ones)
            return 0

        lax.fori_loop(0, CH, body, 0)

    pltpu.sync_copy(lh, sh.at[iotav.at[0]], add=True)
    plsc.subcore_barrier()

    @pl.when(s < NPR // (8 * NC))
    def _():
        base = c * (NPR // NC) + s * 8
        pltpu.sync_copy(sh.at[pl.ds(base, 8)], dbuf)
        magic = jnp.full((16,), 0x5F3759DF, jnp.int32)
        for r in range(8):
            for k in range(8):
                d = dbuf[r, pl.ds(k * 16, 16)] + 1.0
                i = magic - lax.shift_right_logical(
                    plsc.bitcast(d, jnp.int32), 1)
                y = plsc.bitcast(i, jnp.float32)
                for _ in range(3):
                    y = y * (1.5 - 0.5 * d * y * y)
                obuf[r, pl.ds(k * 16, 16)] = y
                dbuf[r, pl.ds(k * 16, 16)] = y * y
        pltpu.sync_copy(obuf, dinv_hbm.at[pl.ds(base, 8)])
        pltpu.sync_copy(dbuf, invdeg_hbm.at[pl.ds(base, 8)])


_hist = functools.partial(
    pl.kernel,
    out_type=(
        jax.ShapeDtypeStruct((NPR, 128), jnp.float32),
        jax.ShapeDtypeStruct((NPR, 128), jnp.float32),
    ),
    mesh=plsc.VectorSubcoreMesh(
        core_axis_name="c", subcore_axis_name="s", num_cores=NC,
        num_subcores=NS),
    scratch_types=[
        pltpu.VMEM((CH, CK), jnp.int32),
        pltpu.VMEM((NPR, 128), jnp.float32),
        pltpu.VMEM((1, NPR), jnp.int32),
        pltpu.VMEM((8, 128), jnp.float32),
        pltpu.VMEM((8, 128), jnp.float32),
        pltpu.VMEM_SHARED((NPR, 128), jnp.float32),
    ],
    compiler_params=pltpu.CompilerParams(needs_layout_passes=False),
)(_hist_body)


NBUF = 2


def _scat_body(hn_hbm, row_hbm, col_hbm, out_hbm, idxr, idxc, buf, acc, *sems):
    c = lax.axis_index("c")
    s = lax.axis_index("s")
    wid = s * NC + c

    pltpu.sync_copy(row_hbm.at[pl.ds(wid * EPW, EPW)], idxr)
    pltpu.sync_copy(col_hbm.at[wid], idxc)

    _zero_rows(buf, CK)
    for k in range(RPT // CK):
        pltpu.sync_copy(buf.at[pl.ds(0, CK)],
                        acc.at[pl.ds(s * RPT + k * CK, CK)])
    plsc.subcore_barrier()

    def _gather(j, b):
        return pltpu.make_async_copy(
            hn_hbm.at[idxr.at[pl.ds(j * CK, CK)]], buf.at[pl.ds(b * CK, CK)],
            sems[b])

    for b in range(NBUF):
        _gather(b, b).start()

    def group(g, _):
        for b in range(NBUF):
            j = g * NBUF + b
            _gather(j, b).wait()
            pltpu.sync_copy(buf.at[pl.ds(b * CK, CK)], acc.at[idxc.at[j]],
                            add=True)

            @pl.when(j + NBUF < CH)
            def _():
                _gather(j + NBUF, b).start()
        return 0

    lax.fori_loop(0, CH // NBUF, group, 0)
    plsc.subcore_barrier()

    pltpu.sync_copy(acc.at[pl.ds(s * RPT, RPT)],
                    out_hbm.at[c, pl.ds(s * RPT, RPT)])


_scat = functools.partial(
    pl.kernel,
    out_type=jax.ShapeDtypeStruct((NC, NP, D), jnp.float32),
    mesh=plsc.VectorSubcoreMesh(
        core_axis_name="c", subcore_axis_name="s", num_cores=NC,
        num_subcores=NS),
    scratch_types=[
        pltpu.VMEM((EPW,), jnp.int32),
        pltpu.VMEM((CH, CK), jnp.int32),
        pltpu.VMEM((NBUF * CK, D), jnp.float32),
        pltpu.VMEM_SHARED((NP, D), jnp.float32),
    ] + [pltpu.SemaphoreType.DMA] * NBUF,
    compiler_params=pltpu.CompilerParams(needs_layout_passes=False),
)(_scat_body)


def _prep_body(x_ref, w1_ref, w2_ref, bsum_ref, dinv_ref, invdeg_ref,
               hn_ref, sb_ref):
    xb = x_ref[...]
    h1 = jnp.dot(xb, w1_ref[...], preferred_element_type=jnp.float32)
    hn_ref[...] = h1 * dinv_ref[...]
    sb_ref[...] = (h1 * invdeg_ref[...]
                   + jnp.dot(xb, w2_ref[...], preferred_element_type=jnp.float32)
                   + bsum_ref[...])


_tc_prep = pl.pallas_call(
    _prep_body,
    grid=(GRID,),
    in_specs=[
        pl.BlockSpec((BLK, D), lambda i: (i, 0)),
        pl.BlockSpec((D, D), lambda i: (0, 0)),
        pl.BlockSpec((D, D), lambda i: (0, 0)),
        pl.BlockSpec((1, D), lambda i: (0, 0)),
        pl.BlockSpec((BLK, 1), lambda i: (i, 0)),
        pl.BlockSpec((BLK, 1), lambda i: (i, 0)),
    ],
    out_specs=(
        pl.BlockSpec((BLK, D), lambda i: (i, 0)),
        pl.BlockSpec((BLK, D), lambda i: (i, 0)),
    ),
    out_shape=(
        jax.ShapeDtypeStruct((NP, D), jnp.float32),
        jax.ShapeDtypeStruct((NP, D), jnp.float32),
    ),
)


def _final_body(acc_ref, dinv_ref, sb_ref, out_ref):
    a = acc_ref[...]
    out_ref[...] = (a[0] + a[1]) * dinv_ref[...] + sb_ref[...]


BLKF = 2000

_tc_final = pl.pallas_call(
    _final_body,
    grid=(N // BLKF,),
    in_specs=[
        pl.BlockSpec((NC, BLKF, D), lambda i: (0, i, 0)),
        pl.BlockSpec((BLKF, 1), lambda i: (i, 0)),
        pl.BlockSpec((BLKF, D), lambda i: (i, 0)),
    ],
    out_specs=pl.BlockSpec((BLKF, D), lambda i: (i, 0)),
    out_shape=jax.ShapeDtypeStruct((N, D), jnp.float32),
)


def kernel(x, edge_index, W1, b1, W2, b2):
    row = edge_index[0]
    col = edge_index[1]

    pad = EP - E
    apad = jnp.arange(pad, dtype=jnp.int32)
    row_p = jnp.concatenate([row, apad % 128])
    col_p = jnp.concatenate([col, N + apad % (NP - N)])
    col_rs = col_p.reshape(NW, CH, CK)

    dinv80, invdeg80 = _hist(col_rs)
    dinv_col = dinv80.reshape(NP, 1)
    invdeg_col = invdeg80.reshape(NP, 1)

    x_pad = jnp.pad(x, ((0, NP - N), (0, 0)))
    bsum = (b1 + b2).reshape(1, D)
    hn, sb = _tc_prep(x_pad, W1, W2, bsum, dinv_col, invdeg_col)

    accs = _scat(hn, row_p, col_rs)
    return _tc_final(accs, dinv_col[:N], sb)


# speedup vs baseline: 1.0378x; 1.0378x over previous
import functools

import jax
import jax.numpy as jnp
from jax import lax
from jax.experimental import pallas as pl
from jax.experimental.pallas import tpu as pltpu
from jax.experimental.pallas import tpu_sc as plsc

N = 10000
D = 128
E = 320000

NC = 2
NS = 16
NW = NC * NS

NP = 10240
NPR = NP // 128
CK = 80
CH = 128
EPW = CH * CK
EP = NW * EPW
RPT = NP // NS
BLK = 2000


def _zero_rows(ref, nrows):
    z = jnp.zeros((16,), jnp.float32)

    def body(r, _):
        for k in range(8):
            ref[r, pl.ds(k * 16, 16)] = z
        return 0

    lax.fori_loop(0, nrows, body, 0)


def _hist_body(col_hbm, out_hbm, colv, lh, iotav, sh):
    c = lax.axis_index("c")
    s = lax.axis_index("s")
    wid = s * NC + c

    pltpu.sync_copy(col_hbm.at[wid], colv)

    _zero_rows(lh, NPR)
    for k in range(NPR // 16):
        iotav[0, pl.ds(k * 16, 16)] = lax.iota(jnp.int32, 16) + 16 * k

    @pl.when(s < NPR // 8)
    def _():
        pltpu.sync_copy(lh.at[pl.ds(s * 8, 8)], sh.at[pl.ds(s * 8, 8)])

    plsc.subcore_barrier()

    ones = jnp.ones((16,), jnp.float32)

    def body(j, _):
        for i in range(CK // 16):
            idx = colv[j, pl.ds(i * 16, 16)]
            plsc.addupdate_scatter(
                lh,
                [lax.shift_right_logical(idx, 7), lax.bitwise_and(idx, 127)],
                ones)
        return 0

    lax.fori_loop(0, CH, body, 0)

    pltpu.sync_copy(lh, sh.at[iotav.at[0]], add=True)
    plsc.subcore_barrier()

    @pl.when(s < NPR // 8)
    def _():
        pltpu.sync_copy(sh.at[pl.ds(s * 8, 8)], out_hbm.at[c, pl.ds(s * 8, 8)])


_hist = functools.partial(
    pl.kernel,
    out_type=jax.ShapeDtypeStruct((NC, NPR, 128), jnp.float32),
    mesh=plsc.VectorSubcoreMesh(
        core_axis_name="c", subcore_axis_name="s", num_cores=NC,
        num_subcores=NS),
    scratch_types=[
        pltpu.VMEM((CH, CK), jnp.int32),
        pltpu.VMEM((NPR, 128), jnp.float32),
        pltpu.VMEM((1, NPR), jnp.int32),
        pltpu.VMEM_SHARED((NPR, 128), jnp.float32),
    ],
    compiler_params=pltpu.CompilerParams(needs_layout_passes=False),
)(_hist_body)


NBUF = 2


def _scat_body(hn_hbm, row_hbm, col_hbm, out_hbm, idxr, idxc, buf, acc, *sems):
    c = lax.axis_index("c")
    s = lax.axis_index("s")
    wid = s * NC + c

    pltpu.sync_copy(row_hbm.at[pl.ds(wid * EPW, EPW)], idxr)
    pltpu.sync_copy(col_hbm.at[wid], idxc)

    _zero_rows(buf, CK)
    for k in range(RPT // CK):
        pltpu.sync_copy(buf.at[pl.ds(0, CK)],
                        acc.at[pl.ds(s * RPT + k * CK, CK)])
    plsc.subcore_barrier()

    def _gather(j, b):
        return pltpu.make_async_copy(
            hn_hbm.at[idxr.at[pl.ds(j * CK, CK)]], buf.at[pl.ds(b * CK, CK)],
            sems[b])

    for b in range(NBUF):
        _gather(b, b).start()

    def group(g, _):
        for b in range(NBUF):
            j = g * NBUF + b
            _gather(j, b).wait()
            pltpu.sync_copy(buf.at[pl.ds(b * CK, CK)], acc.at[idxc.at[j]],
                            add=True)

            @pl.when(j + NBUF < CH)
            def _():
                _gather(j + NBUF, b).start()
        return 0

    lax.fori_loop(0, CH // NBUF, group, 0)
    plsc.subcore_barrier()

    pltpu.sync_copy(acc.at[pl.ds(s * RPT, RPT)],
                    out_hbm.at[c, pl.ds(s * RPT, RPT)])


_scat = functools.partial(
    pl.kernel,
    out_type=jax.ShapeDtypeStruct((NC, NP, D), jnp.float32),
    mesh=plsc.VectorSubcoreMesh(
        core_axis_name="c", subcore_axis_name="s", num_cores=NC,
        num_subcores=NS),
    scratch_types=[
        pltpu.VMEM((EPW,), jnp.int32),
        pltpu.VMEM((CH, CK), jnp.int32),
        pltpu.VMEM((NBUF * CK, D), jnp.float32),
        pltpu.VMEM_SHARED((NP, D), jnp.float32),
    ] + [pltpu.SemaphoreType.DMA] * NBUF,
    compiler_params=pltpu.CompilerParams(needs_layout_passes=False),
)(_scat_body)


def _deg_body(hist_ref, dinv_ref, invdeg_ref):
    h = hist_ref[...]
    deg = h[0] + h[1] + 1.0
    dinv_ref[...] = lax.rsqrt(deg)
    invdeg_ref[...] = 1.0 / deg


_tc_deg = pl.pallas_call(
    _deg_body,
    out_shape=(
        jax.ShapeDtypeStruct((NPR, 128), jnp.float32),
        jax.ShapeDtypeStruct((NPR, 128), jnp.float32),
    ),
)


def _prep_body(x_ref, w1_ref, w2_ref, bsum_ref, dinv_ref, invdeg_ref,
               hn_ref, sb_ref):
    xb = x_ref[...]
    h1 = jnp.dot(xb, w1_ref[...], preferred_element_type=jnp.float32)
    hn_ref[...] = h1 * dinv_ref[...]
    sb_ref[...] = (h1 * invdeg_ref[...]
                   + jnp.dot(xb, w2_ref[...], preferred_element_type=jnp.float32)
                   + bsum_ref[...])


_tc_prep = pl.pallas_call(
    _prep_body,
    grid=(N // BLK,),
    in_specs=[
        pl.BlockSpec((BLK, D), lambda i: (i, 0)),
        pl.BlockSpec((D, D), lambda i: (0, 0)),
        pl.BlockSpec((D, D), lambda i: (0, 0)),
        pl.BlockSpec((1, D), lambda i: (0, 0)),
        pl.BlockSpec((BLK, 1), lambda i: (i, 0)),
        pl.BlockSpec((BLK, 1), lambda i: (i, 0)),
    ],
    out_specs=(
        pl.BlockSpec((BLK, D), lambda i: (i, 0)),
        pl.BlockSpec((BLK, D), lambda i: (i, 0)),
    ),
    out_shape=(
        jax.ShapeDtypeStruct((N, D), jnp.float32),
        jax.ShapeDtypeStruct((N, D), jnp.float32),
    ),
)


def _final_body(acc_ref, dinv_ref, sb_ref, out_ref):
    a = acc_ref[...]
    out_ref[...] = (a[0] + a[1]) * dinv_ref[...] + sb_ref[...]


BLKF = 2000

_tc_final = pl.pallas_call(
    _final_body,
    grid=(N // BLKF,),
    in_specs=[
        pl.BlockSpec((NC, BLKF, D), lambda i: (0, i, 0)),
        pl.BlockSpec((BLKF, 1), lambda i: (i, 0)),
        pl.BlockSpec((BLKF, D), lambda i: (i, 0)),
    ],
    out_specs=pl.BlockSpec((BLKF, D), lambda i: (i, 0)),
    out_shape=jax.ShapeDtypeStruct((N, D), jnp.float32),
)


def kernel(x, edge_index, W1, b1, W2, b2):
    row = edge_index[0]
    col = edge_index[1]

    pad = EP - E
    apad = jnp.arange(pad, dtype=jnp.int32)
    row_p = jnp.concatenate([row, apad % 128])
    col_p = jnp.concatenate([col, N + apad % (NP - N)])
    col_rs = col_p.reshape(NW, CH, CK)

    hist = _hist(col_rs)
    dinv80, invdeg80 = _tc_deg(hist)
    dinv_col = dinv80.reshape(NP, 1)[:N]
    invdeg_col = invdeg80.reshape(NP, 1)[:N]

    bsum = (b1 + b2).reshape(1, D)
    hn, sb = _tc_prep(x, W1, W2, bsum, dinv_col, invdeg_col)

    accs = _scat(hn, row_p, col_rs)
    return _tc_final(accs, dinv_col, sb)


# speedup vs baseline: 1.0567x; 1.0182x over previous
import functools

import jax
import jax.numpy as jnp
from jax import lax
from jax.experimental import pallas as pl
from jax.experimental.pallas import tpu as pltpu
from jax.experimental.pallas import tpu_sc as plsc

N = 10000
D = 128
E = 320000

NC = 2
NS = 16
NW = NC * NS

NP = 10240
NPR = NP // 128
CK = 80
CH = 125
EPW = CH * CK
RPT = NP // NS
BLK = 2000


def _zero_rows(ref, nrows):
    z = jnp.zeros((16,), jnp.float32)

    def body(r, _):
        for k in range(8):
            ref[r, pl.ds(k * 16, 16)] = z
        return 0

    lax.fori_loop(0, nrows, body, 0)


def _hist_body(col_hbm, out_hbm, colv, lh, iotav, sh):
    c = lax.axis_index("c")
    s = lax.axis_index("s")
    wid = s * NC + c

    pltpu.sync_copy(col_hbm.at[wid], colv)

    _zero_rows(lh, NPR)
    for k in range(NPR // 16):
        iotav[0, pl.ds(k * 16, 16)] = lax.iota(jnp.int32, 16) + 16 * k

    @pl.when(s < NPR // 8)
    def _():
        pltpu.sync_copy(lh.at[pl.ds(s * 8, 8)], sh.at[pl.ds(s * 8, 8)])

    plsc.subcore_barrier()

    ones = jnp.ones((16,), jnp.float32)

    def body(j, _):
        for i in range(CK // 16):
            idx = colv[j, pl.ds(i * 16, 16)]
            plsc.addupdate_scatter(
                lh,
                [lax.shift_right_logical(idx, 7), lax.bitwise_and(idx, 127)],
                ones)
        return 0

    lax.fori_loop(0, CH, body, 0)

    pltpu.sync_copy(lh, sh.at[iotav.at[0]], add=True)
    plsc.subcore_barrier()

    @pl.when(s < NPR // 8)
    def _():
        pltpu.sync_copy(sh.at[pl.ds(s * 8, 8)], out_hbm.at[c, pl.ds(s * 8, 8)])


_hist = functools.partial(
    pl.kernel,
    out_type=jax.ShapeDtypeStruct((NC, NPR, 128), jnp.float32),
    mesh=plsc.VectorSubcoreMesh(
        core_axis_name="c", subcore_axis_name="s", num_cores=NC,
        num_subcores=NS),
    scratch_types=[
        pltpu.VMEM((CH, CK), jnp.int32),
        pltpu.VMEM((NPR, 128), jnp.float32),
        pltpu.VMEM((1, NPR), jnp.int32),
        pltpu.VMEM_SHARED((NPR, 128), jnp.float32),
    ],
    compiler_params=pltpu.CompilerParams(needs_layout_passes=False),
)(_hist_body)


NBUF = 2


def _scat_body(hn_hbm, row_hbm, col_hbm, out_hbm, idxr, idxc, buf, acc, *sems):
    c = lax.axis_index("c")
    s = lax.axis_index("s")
    wid = s * NC + c

    pltpu.sync_copy(row_hbm.at[pl.ds(wid * EPW, EPW)], idxr)
    pltpu.sync_copy(col_hbm.at[wid], idxc)

    _zero_rows(buf, CK)
    for k in range(RPT // CK):
        pltpu.sync_copy(buf.at[pl.ds(0, CK)],
                        acc.at[pl.ds(s * RPT + k * CK, CK)])
    plsc.subcore_barrier()

    def _gather(j, b):
        return pltpu.make_async_copy(
            hn_hbm.at[idxr.at[pl.ds(j * CK, CK)]], buf.at[pl.ds(b * CK, CK)],
            sems[b])

    for b in range(NBUF):
        _gather(b, b).start()

    def group(g, _):
        for b in range(NBUF):
            j = g * NBUF + b
            _gather(j, b).wait()
            pltpu.sync_copy(buf.at[pl.ds(b * CK, CK)], acc.at[idxc.at[j]],
                            add=True)

            @pl.when(j + NBUF < CH)
            def _():
                _gather(j + NBUF, b).start()
        return 0

    lax.fori_loop(0, CH // NBUF, group, 0)
    for j in range(NBUF * (CH // NBUF), CH):
        _gather(j, j % NBUF).wait()
        pltpu.sync_copy(buf.at[pl.ds((j % NBUF) * CK, CK)], acc.at[idxc.at[j]],
                        add=True)
    plsc.subcore_barrier()

    pltpu.sync_copy(acc.at[pl.ds(s * RPT, RPT)],
                    out_hbm.at[c, pl.ds(s * RPT, RPT)])


_scat = functools.partial(
    pl.kernel,
    out_type=jax.ShapeDtypeStruct((NC, NP, D), jnp.float32),
    mesh=plsc.VectorSubcoreMesh(
        core_axis_name="c", subcore_axis_name="s", num_cores=NC,
        num_subcores=NS),
    scratch_types=[
        pltpu.VMEM((EPW,), jnp.int32),
        pltpu.VMEM((CH, CK), jnp.int32),
        pltpu.VMEM((NBUF * CK, D), jnp.float32),
        pltpu.VMEM_SHARED((NP, D), jnp.float32),
    ] + [pltpu.SemaphoreType.DMA] * NBUF,
    compiler_params=pltpu.CompilerParams(needs_layout_passes=False),
)(_scat_body)


def _deg_body(hist_ref, dinv_ref, invdeg_ref):
    h = hist_ref[...]
    deg = h[0] + h[1] + 1.0
    dinv_ref[...] = lax.rsqrt(deg)
    invdeg_ref[...] = 1.0 / deg


_tc_deg = pl.pallas_call(
    _deg_body,
    out_shape=(
        jax.ShapeDtypeStruct((NPR, 128), jnp.float32),
        jax.ShapeDtypeStruct((NPR, 128), jnp.float32),
    ),
)


def _prep_body(x_ref, w1_ref, w2_ref, bsum_ref, dinv_ref, invdeg_ref,
               hn_ref, sb_ref):
    xb = x_ref[...]
    h1 = jnp.dot(xb, w1_ref[...], preferred_element_type=jnp.float32)
    hn_ref[...] = h1 * dinv_ref[...]
    sb_ref[...] = (h1 * invdeg_ref[...]
                   + jnp.dot(xb, w2_ref[...], preferred_element_type=jnp.float32)
                   + bsum_ref[...])


_tc_prep = pl.pallas_call(
    _prep_body,
    grid=(N // BLK,),
    in_specs=[
        pl.BlockSpec((BLK, D), lambda i: (i, 0)),
        pl.BlockSpec((D, D), lambda i: (0, 0)),
        pl.BlockSpec((D, D), lambda i: (0, 0)),
        pl.BlockSpec((1, D), lambda i: (0, 0)),
        pl.BlockSpec((BLK, 1), lambda i: (i, 0)),
        pl.BlockSpec((BLK, 1), lambda i: (i, 0)),
    ],
    out_specs=(
        pl.BlockSpec((BLK, D), lambda i: (i, 0)),
        pl.BlockSpec((BLK, D), lambda i: (i, 0)),
    ),
    out_shape=(
        jax.ShapeDtypeStruct((N, D), jnp.float32),
        jax.ShapeDtypeStruct((N, D), jnp.float32),
    ),
)


def _final_body(acc_ref, dinv_ref, sb_ref, out_ref):
    a = acc_ref[...]
    out_ref[...] = (a[0] + a[1]) * dinv_ref[...] + sb_ref[...]


BLKF = 2000

_tc_final = pl.pallas_call(
    _final_body,
    grid=(N // BLKF,),
    in_specs=[
        pl.BlockSpec((NC, BLKF, D), lambda i: (0, i, 0)),
        pl.BlockSpec((BLKF, 1), lambda i: (i, 0)),
        pl.BlockSpec((BLKF, D), lambda i: (i, 0)),
    ],
    out_specs=pl.BlockSpec((BLKF, D), lambda i: (i, 0)),
    out_shape=jax.ShapeDtypeStruct((N, D), jnp.float32),
)


def kernel(x, edge_index, W1, b1, W2, b2):
    row_p = edge_index[0]
    col_rs = edge_index[1].reshape(NW, CH, CK)

    hist = _hist(col_rs)
    dinv80, invdeg80 = _tc_deg(hist)
    dinv_col = dinv80.reshape(NP, 1)[:N]
    invdeg_col = invdeg80.reshape(NP, 1)[:N]

    bsum = (b1 + b2).reshape(1, D)
    hn, sb = _tc_prep(x, W1, W2, bsum, dinv_col, invdeg_col)

    accs = _scat(hn, row_p, col_rs)
    return _tc_final(accs, dinv_col, sb)


# speedup vs baseline: 1.1401x; 1.0789x over previous
import functools

import jax
import jax.numpy as jnp
from jax import lax
from jax.experimental import pallas as pl
from jax.experimental.pallas import tpu as pltpu
from jax.experimental.pallas import tpu_sc as plsc

N = 10000
D = 128
E = 320000

NC = 2
NS = 16
NW = NC * NS

NP = 10240
NPR = NP // 128
CK = 80
CH = 125
EPW = CH * CK
EW = 10496


def _edge_window(wid):
    wid16 = wid * EPW
    start = lax.min(wid16 - lax.rem(wid16, 128), jnp.int32(E - EW))
    start = pl.multiple_of(start, 128)
    return start, wid16 - start
RPT = NP // NS
BLK = 2000


def _zero_rows(ref, nrows):
    z = jnp.zeros((16,), jnp.float32)

    def body(r, _):
        for k in range(8):
            ref[r, pl.ds(k * 16, 16)] = z
        return 0

    lax.fori_loop(0, nrows, body, 0)


def _hist_body(ei_hbm, out_hbm, eiv, lh, iotav, sh):
    c = lax.axis_index("c")
    s = lax.axis_index("s")
    wid = s * NC + c

    start, r = _edge_window(wid)
    pltpu.sync_copy(ei_hbm.at[:, pl.ds(start, EW)], eiv)

    _zero_rows(lh, NPR)
    for k in range(NPR // 16):
        iotav[0, pl.ds(k * 16, 16)] = lax.iota(jnp.int32, 16) + 16 * k

    @pl.when(s < NPR // 8)
    def _():
        pltpu.sync_copy(lh.at[pl.ds(s * 8, 8)], sh.at[pl.ds(s * 8, 8)])

    plsc.subcore_barrier()

    ones = jnp.ones((16,), jnp.float32)

    def body(q, _):
        idx = eiv[1, pl.ds(r + q * 16, 16)]
        plsc.addupdate_scatter(
            lh,
            [lax.shift_right_logical(idx, 7), lax.bitwise_and(idx, 127)],
            ones)
        return 0

    lax.fori_loop(0, EPW // 16, body, 0)

    pltpu.sync_copy(lh, sh.at[iotav.at[0]], add=True)
    plsc.subcore_barrier()

    @pl.when(s < NPR // 8)
    def _():
        pltpu.sync_copy(sh.at[pl.ds(s * 8, 8)], out_hbm.at[c, pl.ds(s * 8, 8)])


_hist = functools.partial(
    pl.kernel,
    out_type=jax.ShapeDtypeStruct((NC, NPR, 128), jnp.float32),
    mesh=plsc.VectorSubcoreMesh(
        core_axis_name="c", subcore_axis_name="s", num_cores=NC,
        num_subcores=NS),
    scratch_types=[
        pltpu.VMEM((2, EW), jnp.int32),
        pltpu.VMEM((NPR, 128), jnp.float32),
        pltpu.VMEM((1, NPR), jnp.int32),
        pltpu.VMEM_SHARED((NPR, 128), jnp.float32),
    ],
    compiler_params=pltpu.CompilerParams(needs_layout_passes=False),
)(_hist_body)


NBUF = 2


def _scat_body(hn_hbm, ei_hbm, out_hbm, ei2, rslot, cslot, buf, acc, *sems):
    c = lax.axis_index("c")
    s = lax.axis_index("s")
    wid = s * NC + c

    start, r = _edge_window(wid)
    pltpu.sync_copy(ei_hbm.at[:, pl.ds(start, EW)], ei2)

    _zero_rows(buf, CK)
    for k in range(RPT // CK):
        pltpu.sync_copy(buf.at[pl.ds(0, CK)],
                        acc.at[pl.ds(s * RPT + k * CK, CK)])
    plsc.subcore_barrier()

    def _fill(slot, row, j, b):
        for i in range(CK // 16):
            slot[b, pl.ds(i * 16, 16)] = ei2[row,
                                             pl.ds(r + j * CK + i * 16, 16)]

    def _gather(b):
        return pltpu.make_async_copy(
            hn_hbm.at[rslot.at[b]], buf.at[pl.ds(b * CK, CK)], sems[b])

    def _start_gather(j, b):
        _fill(rslot, 0, j, b)
        _gather(b).start()

    def _scatter_sync(j, b):
        _fill(cslot, 1, j, b)
        pltpu.sync_copy(buf.at[pl.ds(b * CK, CK)], acc.at[cslot.at[b]],
                        add=True)

    for b in range(NBUF):
        _start_gather(b, b)

    def group(g, _):
        for b in range(NBUF):
            j = g * NBUF + b
            _gather(b).wait()
            _scatter_sync(j, b)

            @pl.when(j + NBUF < CH)
            def _():
                _start_gather(j + NBUF, b)
        return 0

    lax.fori_loop(0, CH // NBUF, group, 0)
    for j in range(NBUF * (CH // NBUF), CH):
        _gather(j % NBUF).wait()
        _scatter_sync(j, j % NBUF)
    plsc.subcore_barrier()

    pltpu.sync_copy(acc.at[pl.ds(s * RPT, RPT)],
                    out_hbm.at[c, pl.ds(s * RPT, RPT)])


_scat = functools.partial(
    pl.kernel,
    out_type=jax.ShapeDtypeStruct((NC, NP, D), jnp.float32),
    mesh=plsc.VectorSubcoreMesh(
        core_axis_name="c", subcore_axis_name="s", num_cores=NC,
        num_subcores=NS),
    scratch_types=[
        pltpu.VMEM((2, EW), jnp.int32),
        pltpu.VMEM((NBUF, CK), jnp.int32),
        pltpu.VMEM((NBUF, CK), jnp.int32),
        pltpu.VMEM((NBUF * CK, D), jnp.float32),
        pltpu.VMEM_SHARED((NP, D), jnp.float32),
    ] + [pltpu.SemaphoreType.DMA] * NBUF,
    compiler_params=pltpu.CompilerParams(needs_layout_passes=False),
)(_scat_body)


def _deg_body(hist_ref, dinv_ref, invdeg_ref):
    h = hist_ref[...]
    deg = h[0] + h[1] + 1.0
    dinv_ref[...] = lax.rsqrt(deg)
    invdeg_ref[...] = 1.0 / deg


_tc_deg = pl.pallas_call(
    _deg_body,
    out_shape=(
        jax.ShapeDtypeStruct((NPR, 128), jnp.float32),
        jax.ShapeDtypeStruct((NPR, 128), jnp.float32),
    ),
)


def _prep_body(x_ref, w1_ref, w2_ref, bsum_ref, dinv_ref, invdeg_ref,
               hn_ref, sb_ref):
    xb = x_ref[...]
    h1 = jnp.dot(xb, w1_ref[...], preferred_element_type=jnp.float32)
    hn_ref[...] = h1 * dinv_ref[...]
    sb_ref[...] = (h1 * invdeg_ref[...]
                   + jnp.dot(xb, w2_ref[...], preferred_element_type=jnp.float32)
                   + bsum_ref[...])


_tc_prep = pl.pallas_call(
    _prep_body,
    grid=(N // BLK,),
    in_specs=[
        pl.BlockSpec((BLK, D), lambda i: (i, 0)),
        pl.BlockSpec((D, D), lambda i: (0, 0)),
        pl.BlockSpec((D, D), lambda i: (0, 0)),
        pl.BlockSpec((1, D), lambda i: (0, 0)),
        pl.BlockSpec((BLK, 1), lambda i: (i, 0)),
        pl.BlockSpec((BLK, 1), lambda i: (i, 0)),
    ],
    out_specs=(
        pl.BlockSpec((BLK, D), lambda i: (i, 0)),
        pl.BlockSpec((BLK, D), lambda i: (i, 0)),
    ),
    out_shape=(
        jax.ShapeDtypeStruct((N, D), jnp.float32),
        jax.ShapeDtypeStruct((N, D), jnp.float32),
    ),
)


def _final_body(acc_ref, dinv_ref, sb_ref, out_ref):
    a = acc_ref[...]
    out_ref[...] = (a[0] + a[1]) * dinv_ref[...] + sb_ref[...]


BLKF = 2000

_tc_final = pl.pallas_call(
    _final_body,
    grid=(N // BLKF,),
    in_specs=[
        pl.BlockSpec((NC, BLKF, D), lambda i: (0, i, 0)),
        pl.BlockSpec((BLKF, 1), lambda i: (i, 0)),
        pl.BlockSpec((BLKF, D), lambda i: (i, 0)),
    ],
    out_specs=pl.BlockSpec((BLKF, D), lambda i: (i, 0)),
    out_shape=jax.ShapeDtypeStruct((N, D), jnp.float32),
)


def kernel(x, edge_index, W1, b1, W2, b2):
    hist = _hist(edge_index)
    dinv80, invdeg80 = _tc_deg(hist)
    dinv_col = dinv80.reshape(NP, 1)[:N]
    invdeg_col = invdeg80.reshape(NP, 1)[:N]

    bsum = (b1 + b2).reshape(1, D)
    hn, sb = _tc_prep(x, W1, W2, bsum, dinv_col, invdeg_col)

    accs = _scat(hn, edge_index)
    return _tc_final(accs, dinv_col, sb)


# speedup vs baseline: 1.1771x; 1.0325x over previous
import functools

import jax
import jax.numpy as jnp
from jax import lax
from jax.experimental import pallas as pl
from jax.experimental.pallas import tpu as pltpu
from jax.experimental.pallas import tpu_sc as plsc

N = 10000
D = 128
E = 320000

NC = 2
NS = 16
NW = NC * NS

NP = 10240
NPR = NP // 128
CK = 80
CH = 125
EPW = CH * CK
EW = 10496


def _edge_window(wid):
    wid16 = wid * EPW
    start = lax.min(wid16 - lax.rem(wid16, 128), jnp.int32(E - EW))
    start = pl.multiple_of(start, 128)
    return start, wid16 - start
RPT = NP // NS
BLK = 2000


def _zero_rows(ref, nrows):
    z = jnp.zeros((16,), jnp.float32)

    def body(r, _):
        for k in range(8):
            ref[r, pl.ds(k * 16, 16)] = z
        return 0

    lax.fori_loop(0, nrows, body, 0)


def _hist_body(ei_hbm, out_hbm, eiv, lh, iotav, sh):
    c = lax.axis_index("c")
    s = lax.axis_index("s")
    wid = s * NC + c

    start, r = _edge_window(wid)
    pltpu.sync_copy(ei_hbm.at[:, pl.ds(start, EW)], eiv)

    _zero_rows(lh, NPR)
    for k in range(NPR // 16):
        iotav[0, pl.ds(k * 16, 16)] = lax.iota(jnp.int32, 16) + 16 * k

    @pl.when(s < NPR // 8)
    def _():
        pltpu.sync_copy(lh.at[pl.ds(s * 8, 8)], sh.at[pl.ds(s * 8, 8)])

    plsc.subcore_barrier()

    ones = jnp.ones((16,), jnp.float32)

    def body(q, _):
        idx = eiv[1, pl.ds(r + q * 16, 16)]
        plsc.addupdate_scatter(
            lh,
            [lax.shift_right_logical(idx, 7), lax.bitwise_and(idx, 127)],
            ones)
        return 0

    lax.fori_loop(0, EPW // 16, body, 0)

    pltpu.sync_copy(lh, sh.at[iotav.at[0]], add=True)
    plsc.subcore_barrier()

    @pl.when(s < NPR // 8)
    def _():
        pltpu.sync_copy(sh.at[pl.ds(s * 8, 8)], out_hbm.at[c, pl.ds(s * 8, 8)])


_hist = functools.partial(
    pl.kernel,
    out_type=jax.ShapeDtypeStruct((NC, NPR, 128), jnp.float32),
    mesh=plsc.VectorSubcoreMesh(
        core_axis_name="c", subcore_axis_name="s", num_cores=NC,
        num_subcores=NS),
    scratch_types=[
        pltpu.VMEM((2, EW), jnp.int32),
        pltpu.VMEM((NPR, 128), jnp.float32),
        pltpu.VMEM((1, NPR), jnp.int32),
        pltpu.VMEM_SHARED((NPR, 128), jnp.float32),
    ],
    compiler_params=pltpu.CompilerParams(needs_layout_passes=False),
)(_hist_body)


NBUF = 2


def _scat_body(hn_hbm, ei_hbm, out_hbm, ei2, rslot, cslot, buf, acc, *sems):
    c = lax.axis_index("c")
    s = lax.axis_index("s")
    wid = s * NC + c

    start, r = _edge_window(wid)
    pltpu.sync_copy(ei_hbm.at[:, pl.ds(start, EW)], ei2)

    _zero_rows(buf, CK)
    for k in range(RPT // CK):
        pltpu.sync_copy(buf.at[pl.ds(0, CK)],
                        acc.at[pl.ds(s * RPT + k * CK, CK)])
    plsc.subcore_barrier()

    def _fill(slot, row, j, b):
        for i in range(CK // 16):
            slot[b, pl.ds(i * 16, 16)] = ei2[row,
                                             pl.ds(r + j * CK + i * 16, 16)]

    def _gather(b):
        return pltpu.make_async_copy(
            hn_hbm.at[rslot.at[b]], buf.at[pl.ds(b * CK, CK)], sems[b])

    def _start_gather(j, b):
        _fill(rslot, 0, j, b)
        _gather(b).start()

    def _scatter_sync(j, b):
        _fill(cslot, 1, j, b)
        pltpu.sync_copy(buf.at[pl.ds(b * CK, CK)], acc.at[cslot.at[b]],
                        add=True)

    for b in range(NBUF):
        _start_gather(b, b)

    def group(g, _):
        for b in range(NBUF):
            j = g * NBUF + b
            _gather(b).wait()
            _scatter_sync(j, b)

            @pl.when(j + NBUF < CH)
            def _():
                _start_gather(j + NBUF, b)
        return 0

    lax.fori_loop(0, CH // NBUF, group, 0)
    for j in range(NBUF * (CH // NBUF), CH):
        _gather(j % NBUF).wait()
        _scatter_sync(j, j % NBUF)
    plsc.subcore_barrier()

    pltpu.sync_copy(acc.at[pl.ds(s * RPT, RPT)],
                    out_hbm.at[c, pl.ds(s * RPT, RPT)])


_scat = functools.partial(
    pl.kernel,
    out_type=jax.ShapeDtypeStruct((NC, NP, D), jnp.float32),
    mesh=plsc.VectorSubcoreMesh(
        core_axis_name="c", subcore_axis_name="s", num_cores=NC,
        num_subcores=NS),
    scratch_types=[
        pltpu.VMEM((2, EW), jnp.int32),
        pltpu.VMEM((NBUF, CK), jnp.int32),
        pltpu.VMEM((NBUF, CK), jnp.int32),
        pltpu.VMEM((NBUF * CK, D), jnp.float32),
        pltpu.VMEM_SHARED((NP, D), jnp.float32),
    ] + [pltpu.SemaphoreType.DMA] * NBUF,
    compiler_params=pltpu.CompilerParams(needs_layout_passes=False),
)(_scat_body)


def _deg_body(hist_ref, dinv_ref):
    h = hist_ref[...]
    dinv_ref[...] = lax.rsqrt(h[0] + h[1] + 1.0)


_tc_deg = pl.pallas_call(
    _deg_body,
    out_shape=jax.ShapeDtypeStruct((NPR, 128), jnp.float32),
)


def _prep_body(x_ref, w1_ref, w2_ref, bsum_ref, dinv_ref, hn_ref, sb_ref):
    xb = x_ref[...]
    dinv = dinv_ref[...]
    h1 = jnp.dot(xb, w1_ref[...], preferred_element_type=jnp.float32)
    hn_ref[...] = h1 * dinv
    sb_ref[...] = (h1 * (dinv * dinv)
                   + jnp.dot(xb, w2_ref[...], preferred_element_type=jnp.float32)
                   + bsum_ref[...])


_tc_prep = pl.pallas_call(
    _prep_body,
    grid=(N // BLK,),
    in_specs=[
        pl.BlockSpec((BLK, D), lambda i: (i, 0)),
        pl.BlockSpec((D, D), lambda i: (0, 0)),
        pl.BlockSpec((D, D), lambda i: (0, 0)),
        pl.BlockSpec((1, D), lambda i: (0, 0)),
        pl.BlockSpec((BLK, 1), lambda i: (i, 0)),
    ],
    out_specs=(
        pl.BlockSpec((BLK, D), lambda i: (i, 0)),
        pl.BlockSpec((BLK, D), lambda i: (i, 0)),
    ),
    out_shape=(
        jax.ShapeDtypeStruct((N, D), jnp.float32),
        jax.ShapeDtypeStruct((N, D), jnp.float32),
    ),
)


def _final_body(acc_ref, dinv_ref, sb_ref, out_ref):
    a = acc_ref[...]
    out_ref[...] = (a[0] + a[1]) * dinv_ref[...] + sb_ref[...]


BLKF = 2000

_tc_final = pl.pallas_call(
    _final_body,
    grid=(N // BLKF,),
    in_specs=[
        pl.BlockSpec((NC, BLKF, D), lambda i: (0, i, 0)),
        pl.BlockSpec((BLKF, 1), lambda i: (i, 0)),
        pl.BlockSpec((BLKF, D), lambda i: (i, 0)),
    ],
    out_specs=pl.BlockSpec((BLKF, D), lambda i: (i, 0)),
    out_shape=jax.ShapeDtypeStruct((N, D), jnp.float32),
)


def kernel(x, edge_index, W1, b1, W2, b2):
    hist = _hist(edge_index)
    dinv80 = _tc_deg(hist)
    dinv_col = dinv80.reshape(NP, 1)[:N]

    bsum = (b1 + b2).reshape(1, D)
    hn, sb = _tc_prep(x, W1, W2, bsum, dinv_col)

    accs = _scat(hn, edge_index)
    return _tc_final(accs, dinv_col, sb)
